# baseline (speedup 1.0000x reference)
import jax
import jax.numpy as jnp
from jax import lax
from jax.experimental import pallas as pl
from jax.experimental.pallas import tpu as pltpu
from jax.experimental.pallas import tpu_sc as plsc

N_TOKENS = 32768
N_EXPERTS = 64
K = 8
L = 16
NC = 2
NS = 16
NW = NC * NS
ROWS_PER_W = N_TOKENS // NW


def _topk_body(logits_hbm, w_hbm, ids_hbm, logits_v, w_v, ids_v):
    wid = lax.axis_index("s") * NC + lax.axis_index("c")
    in_base = wid * (ROWS_PER_W * N_EXPERTS)
    pltpu.sync_copy(logits_hbm.at[pl.ds(in_base, ROWS_PER_W * N_EXPERTS)],
                    logits_v)

    iota = lax.iota(jnp.int32, L)
    lane_lt8 = iota < K
    gidx = jnp.maximum(iota - K, 0)
    group_ids = [iota + g * L for g in range(4)]

    def merge(av, ai, bv, bi):
        cv = jnp.where(lane_lt8, av, lax.rev(bv, (0,)))
        ci = jnp.where(lane_lt8, ai, lax.rev(bi, (0,)))
        return plsc.sort_key_val(cv, ci, descending=True)

    def one_row(off):
        sv, si = [], []
        for g in range(4):
            v = logits_v[pl.ds(off + g * L, L)]
            k, x = plsc.sort_key_val(v, group_ids[g], descending=True)
            sv.append(k)
            si.append(x)
        mv0, mi0 = merge(sv[0], si[0], sv[1], si[1])
        mv1, mi1 = merge(sv[2], si[2], sv[3], si[3])
        fv, fi = merge(mv0, mi0, mv1, mi1)
        e = jnp.exp(fv - jnp.max(fv))
        denom = jnp.sum(jnp.where(lane_lt8, e, 0.0))
        return e / denom, fi

    @plsc.parallel_loop(0, ROWS_PER_W // 2, unroll=8)
    def two_rows(j):
        wa, ia = one_row(2 * j * N_EXPERTS)
        wb, ib = one_row((2 * j + 1) * N_EXPERTS)
        wb8 = wb.at[gidx].get(mode="promise_in_bounds")
        ib8 = ib.at[gidx].get(mode="promise_in_bounds")
        w_v[pl.ds(j * L, L)] = jnp.where(lane_lt8, wa, wb8)
        ids_v[pl.ds(j * L, L)] = jnp.where(lane_lt8, ia, ib8)

    out_base = wid * (ROWS_PER_W * K)
    pltpu.sync_copy(w_v, w_hbm.at[pl.ds(out_base, ROWS_PER_W * K)])
    pltpu.sync_copy(ids_v, ids_hbm.at[pl.ds(out_base, ROWS_PER_W * K)])


def kernel(hidden_states, router_logits):
    del hidden_states
    fn = pl.kernel(
        _topk_body,
        out_type=(
            jax.ShapeDtypeStruct((N_TOKENS * K,), jnp.float32),
            jax.ShapeDtypeStruct((N_TOKENS * K,), jnp.int32),
        ),
        mesh=plsc.VectorSubcoreMesh(core_axis_name="c", subcore_axis_name="s"),
        compiler_params=pltpu.CompilerParams(needs_layout_passes=False),
        scratch_types=[
            pltpu.VMEM((ROWS_PER_W * N_EXPERTS,), jnp.float32),
            pltpu.VMEM((ROWS_PER_W * K,), jnp.float32),
            pltpu.VMEM((ROWS_PER_W * K,), jnp.int32),
        ],
    )
    w_flat, ids_flat = fn(router_logits.reshape(-1))
    return (w_flat.reshape(N_TOKENS, K),
            ids_flat.reshape(N_TOKENS, K),
            router_logits)


# speedup vs baseline: 1.2454x; 1.2454x over previous
import jax
import jax.numpy as jnp
from jax import lax
from jax.experimental import pallas as pl
from jax.experimental.pallas import tpu as pltpu
from jax.experimental.pallas import tpu_sc as plsc

N_TOKENS = 32768
N_EXPERTS = 64
K = 8
L = 16
NC = 2
NS = 16
NW = NC * NS
ROWS_PER_W = N_TOKENS // NW
RT = 8
TILES_PER_W = ROWS_PER_W // RT
CH_T = 32
N_CH = TILES_PER_W // CH_T


def _topk_body(logits_hbm, w_hbm, ids_hbm, logits_v, w_v, ids_v):
    wid = lax.axis_index("s") * NC + lax.axis_index("c")
    tile_base = wid * TILES_PER_W

    logits_t = logits_hbm.reshape(N_TOKENS // RT, RT, N_EXPERTS)
    w_t = w_hbm.reshape(N_TOKENS // RT, RT, L)
    ids_t = ids_hbm.reshape(N_TOKENS // RT, RT, L)

    iota = lax.iota(jnp.int32, L)
    lane_lt8 = iota < K
    group_ids = [iota + g * L for g in range(4)]

    def merge(av, ai, bv, bi):
        cv = jnp.where(lane_lt8, av, lax.rev(bv, (0,)))
        ci = jnp.where(lane_lt8, ai, lax.rev(bi, (0,)))
        return plsc.sort_key_val(cv, ci, descending=True)

    for c in range(N_CH):
        base = tile_base + c * CH_T
        pltpu.sync_copy(logits_t.at[pl.ds(base, CH_T), :, :], logits_v)

        @plsc.parallel_loop(0, CH_T * RT, unroll=8)
        def one_row(r):
            t = r >> 3
            s = r & 7
            sv, si = [], []
            for g in range(4):
                v = logits_v[t, s, pl.ds(g * L, L)]
                k, x = plsc.sort_key_val(v, group_ids[g], descending=True)
                sv.append(k)
                si.append(x)
            mv0, mi0 = merge(sv[0], si[0], sv[1], si[1])
            mv1, mi1 = merge(sv[2], si[2], sv[3], si[3])
            fv, fi = merge(mv0, mi0, mv1, mi1)
            e = jnp.exp(fv - jnp.max(fv))
            denom = jnp.sum(jnp.where(lane_lt8, e, 0.0))
            w_v[t, s, :] = e / denom
            ids_v[t, s, :] = fi

        pltpu.sync_copy(w_v, w_t.at[pl.ds(base, CH_T), :, :])
        pltpu.sync_copy(ids_v, ids_t.at[pl.ds(base, CH_T), :, :])


def kernel(hidden_states, router_logits):
    del hidden_states
    fn = pl.kernel(
        _topk_body,
        out_type=(
            jax.ShapeDtypeStruct((N_TOKENS, L), jnp.float32),
            jax.ShapeDtypeStruct((N_TOKENS, L), jnp.int32),
        ),
        mesh=plsc.VectorSubcoreMesh(core_axis_name="c", subcore_axis_name="s"),
        compiler_params=pltpu.CompilerParams(needs_layout_passes=False),
        scratch_types=[
            pltpu.VMEM((CH_T, RT, N_EXPERTS), jnp.float32),
            pltpu.VMEM((CH_T, RT, L), jnp.float32),
            pltpu.VMEM((CH_T, RT, L), jnp.int32),
        ],
    )
    w16, ids16 = fn(router_logits)
    return w16[:, :K], ids16[:, :K], router_logits


# speedup vs baseline: 2.3857x; 1.9156x over previous
import jax
import jax.numpy as jnp
from jax import lax
from jax.experimental import pallas as pl
from jax.experimental.pallas import tpu as pltpu
from jax.experimental.pallas import tpu_sc as plsc

N_TOKENS = 32768
N_EXPERTS = 64
K = 8
L = 16
NC = 2
NS = 16
NW = NC * NS
TOK_PER_W = N_TOKENS // NW
GROUPS = TOK_PER_W // L

IDMASK = (1 << 6) - 1

SORT8 = [(0, 1), (2, 3), (4, 5), (6, 7),
         (0, 2), (1, 3), (4, 6), (5, 7),
         (1, 2), (5, 6),
         (0, 4), (1, 5), (2, 6), (3, 7),
         (2, 4), (3, 5),
         (1, 2), (3, 4), (5, 6)]

BITONIC8 = [(0, 4), (1, 5), (2, 6), (3, 7),
            (0, 2), (1, 3), (4, 6), (5, 7),
            (0, 1), (2, 3), (4, 5), (6, 7)]


def _sort8_desc(v):
    for i, j in SORT8:
        hi = jnp.maximum(v[i], v[j])
        lo = jnp.minimum(v[i], v[j])
        v[i], v[j] = hi, lo
    return v


def _merge_top8(a, b):
    m = [jnp.maximum(a[i], b[7 - i]) for i in range(8)]
    for i, j in BITONIC8:
        hi = jnp.maximum(m[i], m[j])
        lo = jnp.minimum(m[i], m[j])
        m[i], m[j] = hi, lo
    return m


def _topk_body(lt_hbm, w_hbm, ids_hbm, logits_v, w_v, ids_v):
    wid = lax.axis_index("s") * NC + lax.axis_index("c")
    t0 = wid * TOK_PER_W
    pltpu.sync_copy(lt_hbm.at[:, pl.ds(t0, TOK_PER_W)], logits_v)

    @plsc.parallel_loop(0, GROUPS)
    def group(i):
        off = i * L
        packed = []
        for e in range(N_EXPERTS):
            v = logits_v[e, pl.ds(off, L)]
            vi = plsc.bitcast(v, jnp.int32)
            vi = (vi & ~IDMASK) | (IDMASK - e)
            packed.append(plsc.bitcast(vi, jnp.float32))
        tops = [_sort8_desc(packed[8 * g:8 * g + 8]) for g in range(8)]
        m01 = _merge_top8(tops[0], tops[1])
        m23 = _merge_top8(tops[2], tops[3])
        m45 = _merge_top8(tops[4], tops[5])
        m67 = _merge_top8(tops[6], tops[7])
        m03 = _merge_top8(m01, m23)
        m47 = _merge_top8(m45, m67)
        top = _merge_top8(m03, m47)

        es = [jnp.exp(top[j] - top[0]) for j in range(1, K)]
        denom = es[0]
        for e in es[1:]:
            denom = denom + e
        inv = 1.0 / (denom + 1.0)
        w_v[0, pl.ds(off, L)] = inv
        for j in range(1, K):
            w_v[j, pl.ds(off, L)] = es[j - 1] * inv
        for j in range(K):
            ti = plsc.bitcast(top[j], jnp.int32)
            ids_v[j, pl.ds(off, L)] = IDMASK - (ti & IDMASK)

    pltpu.sync_copy(w_v, w_hbm.at[:, pl.ds(t0, TOK_PER_W)])
    pltpu.sync_copy(ids_v, ids_hbm.at[:, pl.ds(t0, TOK_PER_W)])


def kernel(hidden_states, router_logits):
    del hidden_states
    fn = pl.kernel(
        _topk_body,
        out_type=(
            jax.ShapeDtypeStruct((K, N_TOKENS), jnp.float32),
            jax.ShapeDtypeStruct((K, N_TOKENS), jnp.int32),
        ),
        mesh=plsc.VectorSubcoreMesh(core_axis_name="c", subcore_axis_name="s"),
        compiler_params=pltpu.CompilerParams(needs_layout_passes=False),
        scratch_types=[
            pltpu.VMEM((N_EXPERTS, TOK_PER_W), jnp.float32),
            pltpu.VMEM((K, TOK_PER_W), jnp.float32),
            pltpu.VMEM((K, TOK_PER_W), jnp.int32),
        ],
    )
    w8, ids8 = fn(router_logits.T)
    return w8.T, ids8.T, router_logits


# speedup vs baseline: 2.9801x; 1.2491x over previous
import jax
import jax.numpy as jnp
from jax import lax
from jax.experimental import pallas as pl
from jax.experimental.pallas import tpu as pltpu
from jax.experimental.pallas import tpu_sc as plsc

N_TOKENS = 32768
N_EXPERTS = 64
K = 8
L = 16
NC = 2
NS = 16
NW = NC * NS
TOK_PER_W = N_TOKENS // NW
GROUPS = TOK_PER_W // L

IDMASK = (1 << 6) - 1

SORT8 = [(0, 1), (2, 3), (4, 5), (6, 7),
         (0, 2), (1, 3), (4, 6), (5, 7),
         (1, 2), (5, 6),
         (0, 4), (1, 5), (2, 6), (3, 7),
         (2, 4), (3, 5),
         (1, 2), (3, 4), (5, 6)]

BITONIC8 = [(0, 4), (1, 5), (2, 6), (3, 7),
            (0, 2), (1, 3), (4, 6), (5, 7),
            (0, 1), (2, 3), (4, 5), (6, 7)]


def _sort8_desc(v):
    for i, j in SORT8:
        hi = jnp.maximum(v[i], v[j])
        lo = jnp.minimum(v[i], v[j])
        v[i], v[j] = hi, lo
    return v


def _merge_top8(a, b):
    m = [jnp.maximum(a[i], b[7 - i]) for i in range(8)]
    for i, j in BITONIC8:
        hi = jnp.maximum(m[i], m[j])
        lo = jnp.minimum(m[i], m[j])
        m[i], m[j] = hi, lo
    return m


def _topk_body(lt_hbm, w_hbm, ids_hbm, lt_out_hbm, logits_v, w_v, ids_v, sem):
    wid = lax.axis_index("s") * NC + lax.axis_index("c")
    t0 = wid * TOK_PER_W
    pltpu.sync_copy(lt_hbm.at[:, pl.ds(t0, TOK_PER_W)], logits_v)
    passthrough = pltpu.make_async_copy(
        logits_v, lt_out_hbm.at[:, pl.ds(t0, TOK_PER_W)], sem)
    passthrough.start()

    @plsc.parallel_loop(0, GROUPS, unroll=2)
    def group(i):
        off = i * L
        packed = []
        for e in range(N_EXPERTS):
            v = logits_v[e, pl.ds(off, L)]
            vi = plsc.bitcast(v, jnp.int32)
            vi = (vi & ~IDMASK) | (IDMASK - e)
            packed.append(plsc.bitcast(vi, jnp.float32))
        tops = [_sort8_desc(packed[8 * g:8 * g + 8]) for g in range(8)]
        m01 = _merge_top8(tops[0], tops[1])
        m23 = _merge_top8(tops[2], tops[3])
        m45 = _merge_top8(tops[4], tops[5])
        m67 = _merge_top8(tops[6], tops[7])
        m03 = _merge_top8(m01, m23)
        m47 = _merge_top8(m45, m67)
        top = _merge_top8(m03, m47)

        es = [jnp.exp(top[j] - top[0]) for j in range(1, K)]
        denom = es[0]
        for e in es[1:]:
            denom = denom + e
        inv = 1.0 / (denom + 1.0)
        w_v[0, pl.ds(off, L)] = inv
        for j in range(1, K):
            w_v[j, pl.ds(off, L)] = es[j - 1] * inv
        for j in range(K):
            ti = plsc.bitcast(top[j], jnp.int32)
            ids_v[j, pl.ds(off, L)] = IDMASK - (ti & IDMASK)

    pltpu.sync_copy(w_v, w_hbm.at[:, pl.ds(t0, TOK_PER_W)])
    pltpu.sync_copy(ids_v, ids_hbm.at[:, pl.ds(t0, TOK_PER_W)])
    passthrough.wait()


def kernel(hidden_states, router_logits):
    del hidden_states
    fn = pl.kernel(
        _topk_body,
        out_type=(
            jax.ShapeDtypeStruct((K, N_TOKENS), jnp.float32),
            jax.ShapeDtypeStruct((K, N_TOKENS), jnp.int32),
            jax.ShapeDtypeStruct((N_EXPERTS, N_TOKENS), jnp.float32),
        ),
        mesh=plsc.VectorSubcoreMesh(core_axis_name="c", subcore_axis_name="s"),
        compiler_params=pltpu.CompilerParams(needs_layout_passes=False),
        scratch_types=[
            pltpu.VMEM((N_EXPERTS, TOK_PER_W), jnp.float32),
            pltpu.VMEM((K, TOK_PER_W), jnp.float32),
            pltpu.VMEM((K, TOK_PER_W), jnp.int32),
            pltpu.SemaphoreType.DMA,
        ],
    )
    w8, ids8, lt_out = fn(router_logits.T)
    return w8.T, ids8.T, lt_out.T
